# baseline (speedup 1.0000x reference)
import functools

import jax
import jax.numpy as jnp
from jax import lax
from jax.experimental import pallas as pl
from jax.experimental.pallas import tpu as pltpu
from jax.experimental.pallas import tpu_sc as plsc

N = 10000
E = 320000
D = 128
G = 64
NC = 2
NS = 16
NW = NC * NS
CH = 128
NBUF = 3
RPT = 81
EPT = CH * RPT
EPAD = NW * EPT
NPAD = 10016
HR = 80
HC = 128
ZR = NPAD // NS


def _elu(v):
    return jnp.where(v > 0, v, jnp.exp(v) - 1.0)


def _dinv_col(degp):
    deg = degp[0] + degp[1] + 1.0
    return lax.rsqrt(deg)


def _deg_body(dst_hbm, z_hbm, out_hbm, dstv, hist, rowidx, degacc):
    c = lax.axis_index("c")
    s = lax.axis_index("s")
    w = c * NS + s
    pltpu.sync_copy(z_hbm.at[pl.ds(0, HR)], hist)
    pltpu.sync_copy(dst_hbm.at[w], dstv)
    for i in range(HR // 16):
        rowidx[pl.ds(i * 16, 16)] = lax.iota(jnp.int32, 16) + (16 * i)

    @pl.when(s == 0)
    def _():
        pltpu.sync_copy(z_hbm.at[pl.ds(0, HR)], degacc)

    plsc.subcore_barrier()

    ones = jnp.ones((16,), jnp.float32)

    def step(i, carry):
        idx = dstv[i >> 3, pl.ds((i & 7) * 16, 16)]
        plsc.addupdate_scatter(hist, [idx >> 7, idx & 127], ones)
        return carry

    lax.fori_loop(0, EPT // 16, step, 0)

    plsc.subcore_barrier()
    pltpu.sync_copy(hist, degacc.at[rowidx], add=True)
    plsc.subcore_barrier()

    @pl.when(s == 0)
    def _():
        pltpu.sync_copy(degacc, out_hbm.at[c])


_deg_call = functools.partial(
    pl.kernel,
    mesh=plsc.VectorSubcoreMesh(core_axis_name="c", subcore_axis_name="s"),
    out_type=jax.ShapeDtypeStruct((NC, HR, HC), jnp.float32),
    scratch_types=[
        pltpu.VMEM((RPT, CH), jnp.int32),
        pltpu.VMEM((HR, HC), jnp.float32),
        pltpu.VMEM((HR,), jnp.int32),
        pltpu.VMEM_SHARED((HR, HC), jnp.float32),
    ],
    compiler_params=pltpu.CompilerParams(needs_layout_passes=False),
)(_deg_body)


def _edge_body(g_hbm, idx_hbm, z_hbm, out_hbm, idxb, rows, gsems, ssems, acc):
    c = lax.axis_index("c")
    s = lax.axis_index("s")
    w = c * NS + s
    pltpu.sync_copy(z_hbm, acc.at[pl.ds(s * ZR, ZR)])

    for b in range(NBUF):
        pltpu.sync_copy(idx_hbm.at[w, b], idxb.at[b])
    plsc.subcore_barrier()

    for b in range(NBUF):
        pltpu.async_copy(g_hbm.at[idxb.at[b, 0]], rows[b], gsems[b])

    def group(i, carry):
        for b in range(NBUF):
            j = i * NBUF + b
            pltpu.make_async_copy(g_hbm.at[idxb.at[b, 0]], rows[b],
                                  gsems[b]).wait()
            pltpu.async_copy(rows[b], acc.at[idxb.at[b, 1]], ssems[b],
                             add=True)
            pltpu.make_async_copy(rows[b], acc.at[idxb.at[b, 1]],
                                  ssems[b]).wait()

            @pl.when(j + NBUF < RPT)
            def _():
                pltpu.sync_copy(idx_hbm.at[w, j + NBUF], idxb.at[b])
                pltpu.async_copy(g_hbm.at[idxb.at[b, 0]], rows[b], gsems[b])
        return carry

    lax.fori_loop(0, RPT // NBUF, group, 0)

    plsc.subcore_barrier()

    @pl.when(s == 0)
    def _():
        pltpu.sync_copy(acc, out_hbm.at[c])


_edge_call = functools.partial(
    pl.kernel,
    mesh=plsc.VectorSubcoreMesh(core_axis_name="c", subcore_axis_name="s"),
    out_type=jax.ShapeDtypeStruct((NC, NPAD, D), jnp.float32),
    scratch_types=[
        pltpu.VMEM((NBUF, 2, CH), jnp.int32),
        [pltpu.VMEM((CH, D), jnp.float32) for _ in range(NBUF)],
        [pltpu.SemaphoreType.DMA for _ in range(NBUF)],
        [pltpu.SemaphoreType.DMA for _ in range(NBUF)],
        pltpu.VMEM_SHARED((NPAD, D), jnp.float32),
    ],
)(_edge_body)


def _tc_mm_body(x_ref, w_ref, o_ref):
    o_ref[...] = jnp.dot(x_ref[...], w_ref[...],
                         preferred_element_type=jnp.float32)


def _tc_scale_body(h0_ref, degp_ref, o_ref):
    o_ref[...] = h0_ref[...] * _dinv_col(degp_ref[...])


def _tc_combine_body(aggp_ref, g1_ref, degp_ref, b_ref, w2_ref, o_ref):
    dinv = _dinv_col(degp_ref[...])
    tot = aggp_ref[0] + aggp_ref[1] + g1_ref[...]
    h = _elu(dinv * tot + b_ref[...])
    g2 = jnp.dot(h, w2_ref[...], preferred_element_type=jnp.float32) * dinv
    row = lax.broadcasted_iota(jnp.int32, (NPAD, 1), 0)
    o_ref[...] = jnp.where(row < N, g2, 0.0)


def _tc_pool_body(aggp_ref, g2_ref, degp_ref, b_ref, batch_ref, o_ref):
    dinv = _dinv_col(degp_ref[...])
    tot = aggp_ref[0] + aggp_ref[1] + g2_ref[...]
    out2 = _elu(dinv * tot + b_ref[...])
    bcol = batch_ref[...]
    onehot = (bcol == lax.broadcasted_iota(jnp.int32, (1, G), 1)
              ).astype(jnp.float32)
    sums = lax.dot_general(onehot, out2, (((0,), (0,)), ((), ())),
                           preferred_element_type=jnp.float32)
    cnt = jnp.sum(onehot, axis=0).reshape(G, 1)
    o_ref[...] = sums / jnp.maximum(cnt, 1.0)


def _tc(body, out_shape):
    return pl.pallas_call(body, out_shape=out_shape)


def kernel(x, edge_index, batch, W1, b1, Wmu, bmu, Wsig, bsig):
    f32 = jnp.float32
    x_pad = jnp.pad(x, ((0, NPAD - N), (0, 0)))
    src_r = jnp.pad(edge_index[0], (0, EPAD - E),
                    constant_values=N).reshape(NW, RPT, CH)
    dst_r = jnp.pad(edge_index[1], (0, EPAD - E),
                    constant_values=N).reshape(NW, RPT, CH)
    idx_r = jnp.stack([src_r, dst_r], axis=2)
    batch_col = jnp.pad(batch, (0, NPAD - N), constant_values=G).reshape(NPAD, 1)
    zeros = jnp.zeros((ZR, D), f32)
    W2 = jnp.concatenate([Wmu, Wsig], axis=1)
    b2 = jnp.concatenate([bmu, bsig]).reshape(1, D)
    b1r = b1.reshape(1, D)

    degp = _deg_call(dst_r, zeros)
    degp = degp.reshape(NC, HR * HC, 1)[:, :NPAD]

    h0 = _tc(_tc_mm_body, jax.ShapeDtypeStruct((NPAD, D), f32))(x_pad, W1)
    g1 = _tc(_tc_scale_body, jax.ShapeDtypeStruct((NPAD, D), f32))(h0, degp)

    agg1 = _edge_call(g1, idx_r, zeros)

    g2 = _tc(_tc_combine_body, jax.ShapeDtypeStruct((NPAD, D), f32))(
        agg1, g1, degp, b1r, W2)

    agg2 = _edge_call(g2, idx_r, zeros)

    z = _tc(_tc_pool_body, jax.ShapeDtypeStruct((G, D), f32))(
        agg2, g2, degp, b2, batch_col)

    return (z[:, : D // 2], z[:, D // 2:])


# speedup vs baseline: 1.4496x; 1.4496x over previous
import functools

import jax
import jax.numpy as jnp
from jax import lax
from jax.experimental import pallas as pl
from jax.experimental.pallas import tpu as pltpu
from jax.experimental.pallas import tpu_sc as plsc

N = 10000
E = 320000
D = 128
G = 64
NC = 2
NS = 16
NW = NC * NS
CH = 128
K0 = 120
K1 = 40
KP = K0 + K1
CT = NS * KP
CTP = CT + 128
EPAD = CTP * CH
NPAD = 10016
HR = 80
HC = 128
EPTD = EPAD // NW
RPTD = EPTD // CH
ZR = NPAD // NS


def _elu(v):
    return jnp.where(v > 0, v, jnp.exp(v) - 1.0)


def _dinv_col(degp):
    deg = degp[0] + degp[1] + 1.0
    return lax.rsqrt(deg)


def _deg_body(dst_hbm, z_hbm, out_hbm, dstv, hist, rowidx, degacc):
    c = lax.axis_index("c")
    s = lax.axis_index("s")
    w = c * NS + s
    pltpu.sync_copy(z_hbm.at[pl.ds(0, HR)], hist)
    pltpu.sync_copy(dst_hbm.at[w], dstv)
    for i in range(HR // 16):
        rowidx[pl.ds(i * 16, 16)] = lax.iota(jnp.int32, 16) + (16 * i)

    @pl.when(s == 0)
    def _():
        pltpu.sync_copy(z_hbm.at[pl.ds(0, HR)], degacc)

    plsc.subcore_barrier()

    ones = jnp.ones((16,), jnp.float32)

    def step(i, carry):
        idx = dstv[i >> 3, pl.ds((i & 7) * 16, 16)]
        plsc.addupdate_scatter(hist, [idx >> 7, idx & 127], ones)
        return carry

    lax.fori_loop(0, EPTD // 16, step, 0)

    plsc.subcore_barrier()
    pltpu.sync_copy(hist, degacc.at[rowidx], add=True)
    plsc.subcore_barrier()

    @pl.when(s == 0)
    def _():
        pltpu.sync_copy(degacc, out_hbm.at[c])


_deg_call = functools.partial(
    pl.kernel,
    mesh=plsc.VectorSubcoreMesh(core_axis_name="c", subcore_axis_name="s"),
    out_type=jax.ShapeDtypeStruct((NC, HR, HC), jnp.float32),
    scratch_types=[
        pltpu.VMEM((RPTD, CH), jnp.int32),
        pltpu.VMEM((HR, HC), jnp.float32),
        pltpu.VMEM((HR,), jnp.int32),
        pltpu.VMEM_SHARED((HR, HC), jnp.float32),
    ],
    compiler_params=pltpu.CompilerParams(needs_layout_passes=False),
)(_deg_body)


def _edge_body(g_hbm, idx_hbm, z_hbm, out_hbm, idxb, rows, sem, acc):
    c = lax.axis_index("c")
    s = lax.axis_index("s")
    start = s * KP + c * K0
    cnt = K0 - (K0 - K1) * c
    pltpu.sync_copy(z_hbm, acc.at[pl.ds(s * ZR, ZR)])
    pltpu.sync_copy(idx_hbm.at[pl.ds(start, K0)], idxb)
    plsc.subcore_barrier()

    def step(j, carry):
        pltpu.async_copy(g_hbm.at[idxb.at[j, 0]], rows, sem).wait()
        pltpu.sync_copy(rows, acc.at[idxb.at[j, 1]], add=True)
        return carry

    lax.fori_loop(0, cnt, step, 0)

    plsc.subcore_barrier()

    @pl.when(s == 0)
    def _():
        pltpu.sync_copy(acc, out_hbm.at[c])


_edge_call = functools.partial(
    pl.kernel,
    mesh=plsc.VectorSubcoreMesh(core_axis_name="c", subcore_axis_name="s"),
    out_type=jax.ShapeDtypeStruct((NC, NPAD, D), jnp.float32),
    scratch_types=[
        pltpu.VMEM((K0, 2, CH), jnp.int32),
        pltpu.VMEM((CH, D), jnp.float32),
        pltpu.SemaphoreType.DMA,
        pltpu.VMEM_SHARED((NPAD, D), jnp.float32),
    ],
)(_edge_body)


def _tc_mm_body(x_ref, w_ref, o_ref):
    o_ref[...] = jnp.dot(x_ref[...], w_ref[...],
                         preferred_element_type=jnp.float32)


def _tc_scale_body(h0_ref, degp_ref, o_ref):
    o_ref[...] = h0_ref[...] * _dinv_col(degp_ref[...])


def _tc_combine_body(aggp_ref, g1_ref, degp_ref, b_ref, w2_ref, o_ref):
    dinv = _dinv_col(degp_ref[...])
    tot = aggp_ref[0] + aggp_ref[1] + g1_ref[...]
    h = _elu(dinv * tot + b_ref[...])
    g2 = jnp.dot(h, w2_ref[...], preferred_element_type=jnp.float32) * dinv
    row = lax.broadcasted_iota(jnp.int32, (NPAD, 1), 0)
    o_ref[...] = jnp.where(row < N, g2, 0.0)


def _tc_pool_body(aggp_ref, g2_ref, degp_ref, b_ref, batch_ref, o_ref):
    dinv = _dinv_col(degp_ref[...])
    tot = aggp_ref[0] + aggp_ref[1] + g2_ref[...]
    out2 = _elu(dinv * tot + b_ref[...])
    bcol = batch_ref[...]
    onehot = (bcol == lax.broadcasted_iota(jnp.int32, (1, G), 1)
              ).astype(jnp.float32)
    sums = lax.dot_general(onehot, out2, (((0,), (0,)), ((), ())),
                           preferred_element_type=jnp.float32)
    cnt = jnp.sum(onehot, axis=0).reshape(G, 1)
    o_ref[...] = sums / jnp.maximum(cnt, 1.0)


def _tc(body, out_shape):
    return pl.pallas_call(body, out_shape=out_shape)


def kernel(x, edge_index, batch, W1, b1, Wmu, bmu, Wsig, bsig):
    f32 = jnp.float32
    x_pad = jnp.pad(x, ((0, NPAD - N), (0, 0)))
    src_r = jnp.pad(edge_index[0], (0, EPAD - E), constant_values=N)
    dst_r = jnp.pad(edge_index[1], (0, EPAD - E), constant_values=N)
    idx_r = jnp.stack([src_r.reshape(CTP, CH),
                       dst_r.reshape(CTP, CH)], axis=1)
    dst_d = dst_r.reshape(NW, RPTD, CH)
    batch_col = jnp.pad(batch, (0, NPAD - N), constant_values=G).reshape(NPAD, 1)
    zeros = jnp.zeros((ZR, D), f32)
    W2 = jnp.concatenate([Wmu, Wsig], axis=1)
    b2 = jnp.concatenate([bmu, bsig]).reshape(1, D)
    b1r = b1.reshape(1, D)

    degp = _deg_call(dst_d, zeros)
    degp = degp.reshape(NC, HR * HC, 1)[:, :NPAD]

    h0 = _tc(_tc_mm_body, jax.ShapeDtypeStruct((NPAD, D), f32))(x_pad, W1)
    g1 = _tc(_tc_scale_body, jax.ShapeDtypeStruct((NPAD, D), f32))(h0, degp)

    agg1 = _edge_call(g1, idx_r, zeros)

    g2 = _tc(_tc_combine_body, jax.ShapeDtypeStruct((NPAD, D), f32))(
        agg1, g1, degp, b1r, W2)

    agg2 = _edge_call(g2, idx_r, zeros)

    z = _tc(_tc_pool_body, jax.ShapeDtypeStruct((G, D), f32))(
        agg2, g2, degp, b2, batch_col)

    return (z[:, : D // 2], z[:, D // 2:])
